# speedup vs baseline: 21.0924x; 21.0924x over previous
import jax
import jax.numpy as jnp
from jax.experimental import pallas as pl
from jax.experimental.pallas import tpu as pltpu

B = 8
N = 16384
NB = 512


def _fps_body(xt_ref, fi_ref, out_ref, mind_ref):
    xx = xt_ref[0]
    xy = xt_ref[1]
    xz = xt_ref[2]
    lane = jax.lax.broadcasted_iota(jnp.int32, (B, N), 1)
    col = jax.lax.broadcasted_iota(jnp.int32, (B, NB), 1)
    mind_ref[...] = jnp.full((B, N), jnp.inf, dtype=jnp.float32)
    out_ref[...] = jnp.zeros((B, NB), dtype=jnp.int32)

    def step(t, idx):
        out_ref[...] = jnp.where(col == t, jnp.broadcast_to(idx, (B, NB)),
                                 out_ref[...])
        onehot = lane == idx
        cx = jnp.sum(jnp.where(onehot, xx, 0.0), axis=1, keepdims=True)
        cy = jnp.sum(jnp.where(onehot, xy, 0.0), axis=1, keepdims=True)
        cz = jnp.sum(jnp.where(onehot, xz, 0.0), axis=1, keepdims=True)
        dx = xx - cx
        dy = xy - cy
        dz = xz - cz
        dist = jnp.sqrt(dx * dx + dy * dy + dz * dz)
        md = jnp.minimum(mind_ref[...], dist)
        mind_ref[...] = md
        m = jnp.max(md, axis=1, keepdims=True)
        pick = jnp.min(jnp.where(md == m, lane, N), axis=1, keepdims=True)
        return pick.astype(jnp.int32)

    jax.lax.fori_loop(0, NB, step, fi_ref[...], unroll=False)


def kernel(x, first_index):
    xt = jnp.transpose(x, (2, 0, 1))
    fi = first_index.astype(jnp.int32).reshape(B, 1)
    out = pl.pallas_call(
        _fps_body,
        out_shape=jax.ShapeDtypeStruct((B, NB), jnp.int32),
        in_specs=[
            pl.BlockSpec(memory_space=pltpu.VMEM),
            pl.BlockSpec(memory_space=pltpu.VMEM),
        ],
        out_specs=pl.BlockSpec(memory_space=pltpu.VMEM),
        scratch_shapes=[pltpu.VMEM((B, N), jnp.float32)],
    )(xt, fi)
    return out


# speedup vs baseline: 25.2800x; 1.1985x over previous
import jax
import jax.numpy as jnp
from jax.experimental import pallas as pl
from jax.experimental.pallas import tpu as pltpu

B = 8
N = 16384
NB = 512
W = 1024
NC = N // W


def _fps_body(xq_ref, xg_ref, fi_ref, out_ref, mind_ref):
    lane = jax.lax.broadcasted_iota(jnp.int32, (B, W), 1)
    col = jax.lax.broadcasted_iota(jnp.int32, (B, NB), 1)
    row8 = jax.lax.broadcasted_iota(jnp.int32, (B, 1), 0)
    srow = jax.lax.broadcasted_iota(jnp.int32, (B, 3 * B), 0)
    slane = jax.lax.broadcasted_iota(jnp.int32, (B, 3 * B), 1)
    diag0 = slane == 3 * srow
    diag1 = slane == 3 * srow + 1
    diag2 = slane == 3 * srow + 2
    for c in range(NC):
        mind_ref[c] = jnp.full((B, W), jnp.inf, dtype=jnp.float32)
    out_ref[...] = jnp.zeros((B, NB), dtype=jnp.int32)

    def step(t, idx):
        out_ref[...] = jnp.where(col == t, jnp.broadcast_to(idx, (B, NB)),
                                 out_ref[...])
        stacked = jnp.zeros((B, 3 * B), dtype=jnp.float32)
        for b in range(B):
            pb = jnp.min(jnp.where(row8 == b, idx, N))
            rowv = xg_ref[pb, :]
            stacked = jnp.where(srow == b, rowv[None, :], stacked)
        cx = jnp.sum(jnp.where(diag0, stacked, 0.0), axis=1, keepdims=True)
        cy = jnp.sum(jnp.where(diag1, stacked, 0.0), axis=1, keepdims=True)
        cz = jnp.sum(jnp.where(diag2, stacked, 0.0), axis=1, keepdims=True)

        def chunk(c, carry):
            macc, iacc = carry
            dx = xq_ref[0, c] - cx
            dy = xq_ref[1, c] - cy
            dz = xq_ref[2, c] - cz
            dist = jnp.sqrt(dx * dx + dy * dy + dz * dz)
            mdc = jnp.minimum(mind_ref[c], dist)
            mind_ref[c] = mdc
            better = mdc > macc
            macc = jnp.where(better, mdc, macc)
            iacc = jnp.where(better, c, iacc)
            return macc, iacc

        macc0 = jnp.full((B, W), -jnp.inf, dtype=jnp.float32)
        iacc0 = jnp.zeros((B, W), dtype=jnp.int32)
        macc, iacc = jax.lax.fori_loop(0, NC, chunk, (macc0, iacc0),
                                       unroll=2)
        m = jnp.max(macc, axis=1, keepdims=True)
        flat = iacc * W + lane
        pick = jnp.min(jnp.where(macc == m, flat, N), axis=1, keepdims=True)
        return pick.astype(jnp.int32)

    jax.lax.fori_loop(0, NB, step, fi_ref[...], unroll=False)


def kernel(x, first_index):
    xt = jnp.transpose(x, (2, 0, 1))
    xq = xt.reshape(3, B, NC, W).transpose(0, 2, 1, 3)
    xg = jnp.transpose(x, (1, 0, 2)).reshape(N, 3 * B)
    fi = first_index.astype(jnp.int32).reshape(B, 1)
    out = pl.pallas_call(
        _fps_body,
        out_shape=jax.ShapeDtypeStruct((B, NB), jnp.int32),
        in_specs=[
            pl.BlockSpec(memory_space=pltpu.VMEM),
            pl.BlockSpec(memory_space=pltpu.VMEM),
            pl.BlockSpec(memory_space=pltpu.VMEM),
        ],
        out_specs=pl.BlockSpec(memory_space=pltpu.VMEM),
        scratch_shapes=[pltpu.VMEM((NC, B, W), jnp.float32)],
    )(xq, xg, fi)
    return out


# speedup vs baseline: 34.7143x; 1.3732x over previous
import jax
import jax.numpy as jnp
from jax.experimental import pallas as pl
from jax.experimental.pallas import tpu as pltpu

B = 8
N = 16384
NB = 512
W = 512
NC = N // W


def _fps_body(xq_ref, fi_ref, out_ref, mind_ref):
    lane = jax.lax.broadcasted_iota(jnp.int32, (B, W), 1)
    col = jax.lax.broadcasted_iota(jnp.int32, (B, NB), 1)
    for c in range(NC):
        mind_ref[c] = jnp.full((B, W), jnp.inf, dtype=jnp.float32)
    out_ref[...] = jnp.zeros((B, NB), dtype=jnp.int32)

    fi = fi_ref[...]
    cx0 = jnp.zeros((B, 1), dtype=jnp.float32)
    cy0 = jnp.zeros((B, 1), dtype=jnp.float32)
    cz0 = jnp.zeros((B, 1), dtype=jnp.float32)
    for c in range(NC):
        onehot = (lane + c * W) == fi
        cx0 += jnp.sum(jnp.where(onehot, xq_ref[0, c], 0.0), axis=1,
                       keepdims=True)
        cy0 += jnp.sum(jnp.where(onehot, xq_ref[1, c], 0.0), axis=1,
                       keepdims=True)
        cz0 += jnp.sum(jnp.where(onehot, xq_ref[2, c], 0.0), axis=1,
                       keepdims=True)

    def step(t, carry):
        idx, cx, cy, cz = carry
        out_ref[...] = jnp.where(col == t, jnp.broadcast_to(idx, (B, NB)),
                                 out_ref[...])
        macc = jnp.full((B, W), -jnp.inf, dtype=jnp.float32)
        iacc = jnp.zeros((B, W), dtype=jnp.int32)
        xacc = jnp.zeros((B, W), dtype=jnp.float32)
        yacc = jnp.zeros((B, W), dtype=jnp.float32)
        zacc = jnp.zeros((B, W), dtype=jnp.float32)
        for c in range(NC):
            xc = xq_ref[0, c]
            yc = xq_ref[1, c]
            zc = xq_ref[2, c]
            dx = xc - cx
            dy = yc - cy
            dz = zc - cz
            dist = jnp.sqrt(dx * dx + dy * dy + dz * dz)
            mdc = jnp.minimum(mind_ref[c], dist)
            mind_ref[c] = mdc
            better = mdc > macc
            macc = jnp.where(better, mdc, macc)
            iacc = jnp.where(better, c, iacc)
            xacc = jnp.where(better, xc, xacc)
            yacc = jnp.where(better, yc, yacc)
            zacc = jnp.where(better, zc, zacc)

        flat = iacc * W + lane
        m = jnp.max(macc, axis=1, keepdims=True)
        pick = jnp.min(jnp.where(macc == m, flat, N), axis=1, keepdims=True)
        sel = flat == pick
        ncx = jnp.sum(jnp.where(sel, xacc, 0.0), axis=1, keepdims=True)
        ncy = jnp.sum(jnp.where(sel, yacc, 0.0), axis=1, keepdims=True)
        ncz = jnp.sum(jnp.where(sel, zacc, 0.0), axis=1, keepdims=True)
        return (pick, ncx, ncy, ncz)

    jax.lax.fori_loop(0, NB, step, (fi, cx0, cy0, cz0), unroll=False)


def kernel(x, first_index):
    xt = jnp.transpose(x, (2, 0, 1))
    xq = xt.reshape(3, B, NC, W).transpose(0, 2, 1, 3)
    fi = first_index.astype(jnp.int32).reshape(B, 1)
    out = pl.pallas_call(
        _fps_body,
        out_shape=jax.ShapeDtypeStruct((B, NB), jnp.int32),
        in_specs=[
            pl.BlockSpec(memory_space=pltpu.VMEM),
            pl.BlockSpec(memory_space=pltpu.VMEM),
        ],
        out_specs=pl.BlockSpec(memory_space=pltpu.VMEM),
        scratch_shapes=[pltpu.VMEM((NC, B, W), jnp.float32)],
    )(xq, fi)
    return out


# speedup vs baseline: 35.2715x; 1.0161x over previous
import jax
import jax.numpy as jnp
from jax.experimental import pallas as pl
from jax.experimental.pallas import tpu as pltpu

B = 8
N = 16384
NB = 512
W = 512
NC = N // W
MARGIN = 6e-7


def _fps_body(xq_ref, fi_ref, out_ref, mind_ref, cg_ref):
    lanef = jax.lax.broadcasted_iota(jnp.int32, (B, W), 1).astype(jnp.float32)
    lane = jax.lax.broadcasted_iota(jnp.int32, (B, W), 1)
    col = jax.lax.broadcasted_iota(jnp.int32, (B, NB), 1)
    for c in range(NC):
        mind_ref[c] = jnp.full((B, W), jnp.inf, dtype=jnp.float32)
    out_ref[...] = jnp.zeros((B, NB), dtype=jnp.int32)

    fi = fi_ref[...]
    cx0 = jnp.zeros((B, 1), dtype=jnp.float32)
    cy0 = jnp.zeros((B, 1), dtype=jnp.float32)
    cz0 = jnp.zeros((B, 1), dtype=jnp.float32)
    for c in range(NC):
        onehot = (lane + c * W) == fi
        cx0 += jnp.sum(jnp.where(onehot, xq_ref[0, c], 0.0), axis=1,
                       keepdims=True)
        cy0 += jnp.sum(jnp.where(onehot, xq_ref[1, c], 0.0), axis=1,
                       keepdims=True)
        cz0 += jnp.sum(jnp.where(onehot, xq_ref[2, c], 0.0), axis=1,
                       keepdims=True)

    def step(t, carry):
        idx, cx, cy, cz = carry
        out_ref[...] = jnp.where(col == t, jnp.broadcast_to(idx, (B, NB)),
                                 out_ref[...])
        macc = jnp.full((B, W), -jnp.inf, dtype=jnp.float32)
        iacc = jnp.zeros((B, W), dtype=jnp.float32)
        xacc = jnp.zeros((B, W), dtype=jnp.float32)
        yacc = jnp.zeros((B, W), dtype=jnp.float32)
        zacc = jnp.zeros((B, W), dtype=jnp.float32)
        tie = jnp.zeros((B, W), dtype=jnp.bool_)
        for c in range(NC):
            xc = xq_ref[0, c]
            yc = xq_ref[1, c]
            zc = xq_ref[2, c]
            dx = xc - cx
            dy = yc - cy
            dz = zc - cz
            d2 = dx * dx + dy * dy + dz * dz
            mdc = jnp.minimum(mind_ref[c], d2)
            mind_ref[c] = mdc
            better = mdc > macc
            tie = tie | (better & (mdc <= macc * (1.0 + MARGIN)))
            macc = jnp.where(better, mdc, macc)
            iacc = jnp.where(better, jnp.float32(c), iacc)
            xacc = jnp.where(better, xc, xacc)
            yacc = jnp.where(better, yc, yacc)
            zacc = jnp.where(better, zc, zacc)

        flatf = iacc * jnp.float32(W) + lanef
        m2 = jnp.max(macc, axis=1, keepdims=True)
        eq = macc == m2
        near = macc >= m2 * (1.0 - MARGIN)
        cnt = jnp.sum(jnp.where(near, 1.0, 0.0), axis=1, keepdims=True)
        ties = jnp.sum(jnp.where(tie, 1.0, 0.0), axis=1, keepdims=True)
        cg_ref[0] = jnp.sum(jnp.where(eq, xacc, 0.0), axis=1, keepdims=True)
        cg_ref[1] = jnp.sum(jnp.where(eq, yacc, 0.0), axis=1, keepdims=True)
        cg_ref[2] = jnp.sum(jnp.where(eq, zacc, 0.0), axis=1, keepdims=True)
        cg_ref[3] = jnp.sum(jnp.where(eq, flatf, 0.0), axis=1, keepdims=True)
        bad = jnp.maximum(cnt - 1.0, ties)

        @pl.when(jnp.max(bad) > 0.5)
        def _exact_fallback():
            sacc = jnp.full((B, W), -jnp.inf, dtype=jnp.float32)
            ifac = jnp.zeros((B, W), dtype=jnp.float32)
            xa = jnp.zeros((B, W), dtype=jnp.float32)
            ya = jnp.zeros((B, W), dtype=jnp.float32)
            za = jnp.zeros((B, W), dtype=jnp.float32)
            for c in range(NC):
                sd = jnp.sqrt(mind_ref[c])
                b2 = sd > sacc
                sacc = jnp.where(b2, sd, sacc)
                ifac = jnp.where(b2, jnp.float32(c), ifac)
                xa = jnp.where(b2, xq_ref[0, c], xa)
                ya = jnp.where(b2, xq_ref[1, c], ya)
                za = jnp.where(b2, xq_ref[2, c], za)
            fl = ifac * jnp.float32(W) + lanef
            ms = jnp.max(sacc, axis=1, keepdims=True)
            pf = jnp.min(jnp.where(sacc == ms, fl, jnp.float32(N)), axis=1,
                         keepdims=True)
            sel = fl == pf
            cg_ref[0] = jnp.sum(jnp.where(sel, xa, 0.0), axis=1,
                                keepdims=True)
            cg_ref[1] = jnp.sum(jnp.where(sel, ya, 0.0), axis=1,
                                keepdims=True)
            cg_ref[2] = jnp.sum(jnp.where(sel, za, 0.0), axis=1,
                                keepdims=True)
            cg_ref[3] = pf

        pick = cg_ref[3].astype(jnp.int32)
        return (pick, cg_ref[0], cg_ref[1], cg_ref[2])

    jax.lax.fori_loop(0, NB, step, (fi, cx0, cy0, cz0), unroll=False)


def kernel(x, first_index):
    xt = jnp.transpose(x, (2, 0, 1))
    xq = xt.reshape(3, B, NC, W).transpose(0, 2, 1, 3)
    fi = first_index.astype(jnp.int32).reshape(B, 1)
    out = pl.pallas_call(
        _fps_body,
        out_shape=jax.ShapeDtypeStruct((B, NB), jnp.int32),
        in_specs=[
            pl.BlockSpec(memory_space=pltpu.VMEM),
            pl.BlockSpec(memory_space=pltpu.VMEM),
        ],
        out_specs=pl.BlockSpec(memory_space=pltpu.VMEM),
        scratch_shapes=[
            pltpu.VMEM((NC, B, W), jnp.float32),
            pltpu.VMEM((4, B, 1), jnp.float32),
        ],
    )(xq, fi)
    return out


# speedup vs baseline: 40.3258x; 1.1433x over previous
import jax
import jax.numpy as jnp
from jax.experimental import pallas as pl
from jax.experimental.pallas import tpu as pltpu

B = 8
N = 16384
NB = 512
W = 512
NC = N // W
MARGIN = 6e-7


def _fps_body(xq_ref, fi_ref, out_ref, mind_ref, cg_ref):
    lanef = jax.lax.broadcasted_iota(jnp.int32, (B, W), 1).astype(jnp.float32)
    lane = jax.lax.broadcasted_iota(jnp.int32, (B, W), 1)
    col = jax.lax.broadcasted_iota(jnp.int32, (B, NB), 1)
    for c in range(NC):
        mind_ref[c] = jnp.full((B, W), jnp.inf, dtype=jnp.float32)
    out_ref[...] = jnp.zeros((B, NB), dtype=jnp.int32)

    fi = fi_ref[...]
    cx0 = jnp.zeros((B, 1), dtype=jnp.float32)
    cy0 = jnp.zeros((B, 1), dtype=jnp.float32)
    cz0 = jnp.zeros((B, 1), dtype=jnp.float32)
    for c in range(NC):
        onehot = (lane + c * W) == fi
        cx0 += jnp.sum(jnp.where(onehot, xq_ref[0, c], 0.0), axis=1,
                       keepdims=True)
        cy0 += jnp.sum(jnp.where(onehot, xq_ref[1, c], 0.0), axis=1,
                       keepdims=True)
        cz0 += jnp.sum(jnp.where(onehot, xq_ref[2, c], 0.0), axis=1,
                       keepdims=True)

    def step(t, carry):
        idx, cx, cy, cz = carry
        out_ref[...] = jnp.where(col == t, jnp.broadcast_to(idx, (B, NB)),
                                 out_ref[...])
        macc = jnp.full((B, W), -jnp.inf, dtype=jnp.float32)
        s2acc = jnp.full((B, W), -jnp.inf, dtype=jnp.float32)
        iacc = jnp.zeros((B, W), dtype=jnp.float32)
        xacc = jnp.zeros((B, W), dtype=jnp.float32)
        yacc = jnp.zeros((B, W), dtype=jnp.float32)
        zacc = jnp.zeros((B, W), dtype=jnp.float32)
        for c in range(NC):
            xc = xq_ref[0, c]
            yc = xq_ref[1, c]
            zc = xq_ref[2, c]
            dx = xc - cx
            dy = yc - cy
            dz = zc - cz
            d2 = dx * dx + dy * dy + dz * dz
            mdc = jnp.minimum(mind_ref[c], d2)
            mind_ref[c] = mdc
            s2acc = jnp.maximum(s2acc, jnp.minimum(mdc, macc))
            better = mdc > macc
            macc = jnp.where(better, mdc, macc)
            iacc = jnp.where(better, jnp.float32(c), iacc)
            xacc = jnp.where(better, xc, xacc)
            yacc = jnp.where(better, yc, yacc)
            zacc = jnp.where(better, zc, zacc)

        flatf = iacc * jnp.float32(W) + lanef
        m2 = jnp.max(macc, axis=1, keepdims=True)
        eq = macc == m2
        near = macc >= m2 * (1.0 - MARGIN)
        risk = s2acc >= macc * (1.0 - MARGIN)
        cnt = jnp.sum(jnp.where(near, 1.0, 0.0) + jnp.where(risk, 2.0, 0.0),
                      axis=1, keepdims=True)
        cg_ref[0] = jnp.sum(jnp.where(eq, xacc, 0.0), axis=1, keepdims=True)
        cg_ref[1] = jnp.sum(jnp.where(eq, yacc, 0.0), axis=1, keepdims=True)
        cg_ref[2] = jnp.sum(jnp.where(eq, zacc, 0.0), axis=1, keepdims=True)
        cg_ref[3] = jnp.sum(jnp.where(eq, flatf, 0.0), axis=1, keepdims=True)

        @pl.when(jnp.max(cnt) > 1.5)
        def _exact_fallback():
            def fb_chunk(c, fbc):
                sacc, ifac, xa, ya, za = fbc
                sd = jnp.sqrt(mind_ref[c])
                b2 = sd > sacc
                cf = c.astype(jnp.float32) * jnp.float32(W) + lanef
                return (jnp.where(b2, sd, sacc),
                        jnp.where(b2, cf, ifac),
                        jnp.where(b2, xq_ref[0, c], xa),
                        jnp.where(b2, xq_ref[1, c], ya),
                        jnp.where(b2, xq_ref[2, c], za))

            ninf = jnp.full((B, W), -jnp.inf, dtype=jnp.float32)
            zero = jnp.zeros((B, W), dtype=jnp.float32)
            sacc, fl, xa, ya, za = jax.lax.fori_loop(
                0, NC, fb_chunk, (ninf, zero, zero, zero, zero))
            ms = jnp.max(sacc, axis=1, keepdims=True)
            pf = jnp.min(jnp.where(sacc == ms, fl, jnp.float32(N)), axis=1,
                         keepdims=True)
            sel = fl == pf
            cg_ref[0] = jnp.sum(jnp.where(sel, xa, 0.0), axis=1,
                                keepdims=True)
            cg_ref[1] = jnp.sum(jnp.where(sel, ya, 0.0), axis=1,
                                keepdims=True)
            cg_ref[2] = jnp.sum(jnp.where(sel, za, 0.0), axis=1,
                                keepdims=True)
            cg_ref[3] = pf

        pick = cg_ref[3].astype(jnp.int32)
        return (pick, cg_ref[0], cg_ref[1], cg_ref[2])

    jax.lax.fori_loop(0, NB, step, (fi, cx0, cy0, cz0), unroll=False)


def kernel(x, first_index):
    xt = jnp.transpose(x, (2, 0, 1))
    xq = xt.reshape(3, B, NC, W).transpose(0, 2, 1, 3)
    fi = first_index.astype(jnp.int32).reshape(B, 1)
    out = pl.pallas_call(
        _fps_body,
        out_shape=jax.ShapeDtypeStruct((B, NB), jnp.int32),
        in_specs=[
            pl.BlockSpec(memory_space=pltpu.VMEM),
            pl.BlockSpec(memory_space=pltpu.VMEM),
        ],
        out_specs=pl.BlockSpec(memory_space=pltpu.VMEM),
        scratch_shapes=[
            pltpu.VMEM((NC, B, W), jnp.float32),
            pltpu.VMEM((4, B, 1), jnp.float32),
        ],
    )(xq, fi)
    return out


# speedup vs baseline: 40.5232x; 1.0049x over previous
import jax
import jax.numpy as jnp
from jax.experimental import pallas as pl
from jax.experimental.pallas import tpu as pltpu

B = 8
N = 16384
NB = 512
W = 512
NC = N // W
MARGIN = 6e-7


def _fps_body(xq_ref, fi_ref, out_ref, mind_ref, cg_ref):
    lanef = jax.lax.broadcasted_iota(jnp.int32, (B, W), 1).astype(jnp.float32)
    lane = jax.lax.broadcasted_iota(jnp.int32, (B, W), 1)
    col = jax.lax.broadcasted_iota(jnp.int32, (B, NB), 1)
    for c in range(NC):
        mind_ref[c] = jnp.full((B, W), jnp.inf, dtype=jnp.float32)
    out_ref[...] = jnp.zeros((B, NB), dtype=jnp.int32)

    fi = fi_ref[...]
    cx0 = jnp.zeros((B, 1), dtype=jnp.float32)
    cy0 = jnp.zeros((B, 1), dtype=jnp.float32)
    cz0 = jnp.zeros((B, 1), dtype=jnp.float32)
    for c in range(NC):
        onehot = (lane + c * W) == fi
        cx0 += jnp.sum(jnp.where(onehot, xq_ref[0, c], 0.0), axis=1,
                       keepdims=True)
        cy0 += jnp.sum(jnp.where(onehot, xq_ref[1, c], 0.0), axis=1,
                       keepdims=True)
        cz0 += jnp.sum(jnp.where(onehot, xq_ref[2, c], 0.0), axis=1,
                       keepdims=True)

    def step(t, carry):
        idx, cx, cy, cz = carry
        out_ref[...] = jnp.where(col == t, jnp.broadcast_to(idx, (B, NB)),
                                 out_ref[...])
        macc = jnp.full((B, W), -jnp.inf, dtype=jnp.float32)
        s2acc = jnp.full((B, W), -jnp.inf, dtype=jnp.float32)
        iacc = jnp.zeros((B, W), dtype=jnp.float32)
        xacc = jnp.zeros((B, W), dtype=jnp.float32)
        yacc = jnp.zeros((B, W), dtype=jnp.float32)
        zacc = jnp.zeros((B, W), dtype=jnp.float32)
        for c in range(NC):
            xc = xq_ref[0, c]
            yc = xq_ref[1, c]
            zc = xq_ref[2, c]
            dx = xc - cx
            dy = yc - cy
            dz = zc - cz
            d2 = dx * dx + dy * dy + dz * dz
            mdc = jnp.minimum(mind_ref[c], d2)
            mind_ref[c] = mdc
            s2acc = jnp.maximum(s2acc, jnp.minimum(mdc, macc))
            better = mdc > macc
            macc = jnp.where(better, mdc, macc)
            iacc = jnp.where(better, jnp.float32(c), iacc)
            xacc = jnp.where(better, xc, xacc)
            yacc = jnp.where(better, yc, yacc)
            zacc = jnp.where(better, zc, zacc)

        flatf = iacc * jnp.float32(W) + lanef
        m2 = jnp.max(macc, axis=1, keepdims=True)
        eq = macc == m2
        near = macc >= m2 * (1.0 - MARGIN)
        risk = s2acc >= macc * (1.0 - MARGIN)
        cnt = jnp.sum(jnp.where(near, 1.0, 0.0) + jnp.where(risk, 2.0, 0.0),
                      axis=1, keepdims=True)
        cg_ref[0] = jnp.sum(jnp.where(eq, xacc, 0.0), axis=1, keepdims=True)
        cg_ref[1] = jnp.sum(jnp.where(eq, yacc, 0.0), axis=1, keepdims=True)
        cg_ref[2] = jnp.sum(jnp.where(eq, zacc, 0.0), axis=1, keepdims=True)
        cg_ref[3] = jnp.sum(jnp.where(eq, flatf, 0.0), axis=1, keepdims=True)

        @pl.when(jnp.max(cnt) > 1.5)
        def _exact_fallback():
            def fb_chunk(c, fbc):
                sacc, ifac, xa, ya, za = fbc
                sd = jnp.sqrt(mind_ref[c])
                b2 = sd > sacc
                cf = c.astype(jnp.float32) * jnp.float32(W) + lanef
                return (jnp.where(b2, sd, sacc),
                        jnp.where(b2, cf, ifac),
                        jnp.where(b2, xq_ref[0, c], xa),
                        jnp.where(b2, xq_ref[1, c], ya),
                        jnp.where(b2, xq_ref[2, c], za))

            ninf = jnp.full((B, W), -jnp.inf, dtype=jnp.float32)
            zero = jnp.zeros((B, W), dtype=jnp.float32)
            sacc, fl, xa, ya, za = jax.lax.fori_loop(
                0, NC, fb_chunk, (ninf, zero, zero, zero, zero))
            ms = jnp.max(sacc, axis=1, keepdims=True)
            pf = jnp.min(jnp.where(sacc == ms, fl, jnp.float32(N)), axis=1,
                         keepdims=True)
            sel = fl == pf
            cg_ref[0] = jnp.sum(jnp.where(sel, xa, 0.0), axis=1,
                                keepdims=True)
            cg_ref[1] = jnp.sum(jnp.where(sel, ya, 0.0), axis=1,
                                keepdims=True)
            cg_ref[2] = jnp.sum(jnp.where(sel, za, 0.0), axis=1,
                                keepdims=True)
            cg_ref[3] = pf

        pick = cg_ref[3].astype(jnp.int32)
        return (pick, cg_ref[0], cg_ref[1], cg_ref[2])

    def step2(i, carry):
        return step(2 * i + 1, step(2 * i, carry))

    jax.lax.fori_loop(0, NB // 2, step2, (fi, cx0, cy0, cz0), unroll=False)


def kernel(x, first_index):
    xt = jnp.transpose(x, (2, 0, 1))
    xq = xt.reshape(3, B, NC, W).transpose(0, 2, 1, 3)
    fi = first_index.astype(jnp.int32).reshape(B, 1)
    out = pl.pallas_call(
        _fps_body,
        out_shape=jax.ShapeDtypeStruct((B, NB), jnp.int32),
        in_specs=[
            pl.BlockSpec(memory_space=pltpu.VMEM),
            pl.BlockSpec(memory_space=pltpu.VMEM),
        ],
        out_specs=pl.BlockSpec(memory_space=pltpu.VMEM),
        scratch_shapes=[
            pltpu.VMEM((NC, B, W), jnp.float32),
            pltpu.VMEM((4, B, 1), jnp.float32),
        ],
    )(xq, fi)
    return out
